# Initial kernel scaffold; baseline (speedup 1.0000x reference)
import jax
import jax.numpy as jnp
from jax.experimental import pallas as pl


def kernel(x, emb, pos, ln1_g, ln1_b, Wqk, Wv, Wo, ln2_g, ln2_b, W1, b1, W2, b2, rot, Wcls, bcls):
    raise NotImplementedError("write your pallas kernel here")


# speedup vs baseline: 42.0316x; 42.0316x over previous
import functools

import jax
import jax.numpy as jnp
from jax.experimental import pallas as pl
from jax.experimental.pallas import tpu as pltpu

DIM = 768
DEPTH = 2
NCLASS = 16


def _body(idx_ref, emb_ref, pos_ref, ln1g_ref, ln1b_ref, Wv_ref, Wo_ref,
          ln2g_ref, ln2b_ref, W1_ref, b1_ref, W2_ref, b2_ref,
          Wcls_ref, bcls_ref, out_ref, h_ref):
    l = pl.program_id(0)
    b = pl.program_id(1)

    @pl.when(l == 0)
    def _init():
        h_ref[pl.ds(b, 1), :] = emb_ref[0] + pos_ref[...]

    h = h_ref[pl.ds(b, 1), :]

    def ln(v, g, bb):
        mu = jnp.mean(v, axis=-1, keepdims=True)
        var = jnp.mean((v - mu) ** 2, axis=-1, keepdims=True)
        return (v - mu) / jnp.sqrt(var + 1e-5) * g + bb

    xln = ln(h, ln1g_ref[0], ln1b_ref[0])
    a = jnp.dot(jnp.dot(xln, Wv_ref[0], preferred_element_type=jnp.float32),
                Wo_ref[0], preferred_element_type=jnp.float32)
    h = h + a
    fln = ln(h, ln2g_ref[0], ln2b_ref[0])
    f = jax.nn.gelu(jnp.dot(fln, W1_ref[0], preferred_element_type=jnp.float32)
                    + b1_ref[0])
    f = jnp.dot(f, W2_ref[0], preferred_element_type=jnp.float32) + b2_ref[0]
    h = h + f
    h_ref[pl.ds(b, 1), :] = h

    @pl.when(l == DEPTH - 1)
    def _emit():
        out_ref[pl.ds(b, 1), 0, :] = jnp.dot(
            h, Wcls_ref[...], preferred_element_type=jnp.float32) + bcls_ref[...]


def kernel(x, emb, pos, ln1_g, ln1_b, Wqk, Wv, Wo, ln2_g, ln2_b,
           W1, b1, W2, b2, rot, Wcls, bcls):
    B = x.shape[0]
    idx = x[:, 0]
    emb3 = emb.reshape(emb.shape[0], 1, DIM)
    pos0 = pos[0:1, :]
    r3 = lambda t: t.reshape(DEPTH, 1, -1)

    grid = (DEPTH, B)
    wrow = lambda: pl.BlockSpec((1, 1, DIM), lambda l, b, idx: (l, 0, 0))
    out = pl.pallas_call(
        _body,
        grid_spec=pltpu.PrefetchScalarGridSpec(
            num_scalar_prefetch=1,
            grid=grid,
            in_specs=[
                pl.BlockSpec((1, 1, DIM), lambda l, b, idx: (idx[b], 0, 0)),
                pl.BlockSpec((1, DIM), lambda l, b, idx: (0, 0)),
                wrow(),
                wrow(),
                pl.BlockSpec((1, DIM, DIM), lambda l, b, idx: (l, 0, 0)),
                pl.BlockSpec((1, DIM, DIM), lambda l, b, idx: (l, 0, 0)),
                wrow(),
                wrow(),
                pl.BlockSpec((1, DIM, 4 * DIM), lambda l, b, idx: (l, 0, 0)),
                pl.BlockSpec((1, 1, 4 * DIM), lambda l, b, idx: (l, 0, 0)),
                pl.BlockSpec((1, 4 * DIM, DIM), lambda l, b, idx: (l, 0, 0)),
                wrow(),
                pl.BlockSpec((DIM, NCLASS), lambda l, b, idx: (0, 0)),
                pl.BlockSpec((1, NCLASS), lambda l, b, idx: (0, 0)),
            ],
            out_specs=pl.BlockSpec((B, 1, NCLASS), lambda l, b, idx: (0, 0, 0)),
            scratch_shapes=[pltpu.VMEM((B, DIM), jnp.float32)],
        ),
        out_shape=jax.ShapeDtypeStruct((B, 1, NCLASS), jnp.float32),
    )(idx, emb3, pos0, r3(ln1_g), r3(ln1_b), Wv, Wo, r3(ln2_g), r3(ln2_b),
      W1, r3(b1), W2, r3(b2), Wcls, bcls.reshape(1, NCLASS))
    return out.reshape(B, NCLASS)


# speedup vs baseline: 406.1343x; 9.6626x over previous
import jax
import jax.numpy as jnp
from jax.experimental import pallas as pl
from jax.experimental.pallas import tpu as pltpu

DIM = 768
DEPTH = 2
NCLASS = 16


def _body(idx_ref, emb_any, Wv_any, Wo_any, W1_any, W2_any,
          pos_v, ln1g, ln1b, ln2g, ln2b, b1v, b2v, Wclsv, bclsv,
          out_ref, e_v, Wv_v, Wo_v, W1_v, W2_v, sems):
    B = e_v.shape[0]

    ecp = [pltpu.make_async_copy(emb_any.at[pl.ds(idx_ref[b], 1), :],
                                 e_v.at[pl.ds(b, 1), :], sems.at[b])
           for b in range(B)]
    wcp = []
    for l in range(DEPTH):
        s = B + 4 * l
        wcp.append([
            pltpu.make_async_copy(Wv_any.at[l], Wv_v.at[l], sems.at[s]),
            pltpu.make_async_copy(Wo_any.at[l], Wo_v.at[l], sems.at[s + 1]),
            pltpu.make_async_copy(W1_any.at[l], W1_v.at[l], sems.at[s + 2]),
            pltpu.make_async_copy(W2_any.at[l], W2_v.at[l], sems.at[s + 3]),
        ])
    for c in ecp:
        c.start()
    for cs in wcp:
        for c in cs:
            c.start()

    def ln(v, g, bb):
        mu = jnp.mean(v, axis=-1, keepdims=True)
        var = jnp.mean((v - mu) ** 2, axis=-1, keepdims=True)
        return (v - mu) / jnp.sqrt(var + 1e-5) * g + bb

    for c in ecp:
        c.wait()
    h = e_v[...] + pos_v[...]

    for l in range(DEPTH):
        sl = pl.ds(l, 1)
        xln = ln(h, ln1g[sl, :], ln1b[sl, :])
        wcp[l][0].wait()
        a = jnp.dot(xln, Wv_v[l], preferred_element_type=jnp.float32)
        wcp[l][1].wait()
        a = jnp.dot(a, Wo_v[l], preferred_element_type=jnp.float32)
        h = h + a
        fln = ln(h, ln2g[sl, :], ln2b[sl, :])
        wcp[l][2].wait()
        f = jax.nn.gelu(jnp.dot(fln, W1_v[l], preferred_element_type=jnp.float32)
                        + b1v[sl, :])
        wcp[l][3].wait()
        f = jnp.dot(f, W2_v[l], preferred_element_type=jnp.float32) + b2v[sl, :]
        h = h + f

    out_ref[...] = jnp.dot(h, Wclsv[...],
                           preferred_element_type=jnp.float32) + bclsv[...]


def kernel(x, emb, pos, ln1_g, ln1_b, Wqk, Wv, Wo, ln2_g, ln2_b,
           W1, b1, W2, b2, rot, Wcls, bcls):
    B = x.shape[0]
    idx = x[:, 0]
    pos0 = pos[0:1, :]

    full = lambda shp: pl.BlockSpec(shp, lambda i, idx: tuple(0 for _ in shp))
    any_spec = pl.BlockSpec(memory_space=pl.MemorySpace.ANY)

    out = pl.pallas_call(
        _body,
        grid_spec=pltpu.PrefetchScalarGridSpec(
            num_scalar_prefetch=1,
            grid=(1,),
            in_specs=[
                any_spec,
                any_spec,
                any_spec,
                any_spec,
                any_spec,
                full((1, DIM)),
                full((DEPTH, DIM)),
                full((DEPTH, DIM)),
                full((DEPTH, DIM)),
                full((DEPTH, DIM)),
                full((DEPTH, 4 * DIM)),
                full((DEPTH, DIM)),
                full((DIM, NCLASS)),
                full((1, NCLASS)),
            ],
            out_specs=full((B, NCLASS)),
            scratch_shapes=[
                pltpu.VMEM((B, DIM), jnp.float32),
                pltpu.VMEM((DEPTH, DIM, DIM), jnp.float32),
                pltpu.VMEM((DEPTH, DIM, DIM), jnp.float32),
                pltpu.VMEM((DEPTH, DIM, 4 * DIM), jnp.float32),
                pltpu.VMEM((DEPTH, 4 * DIM, DIM), jnp.float32),
                pltpu.SemaphoreType.DMA((B + 4 * DEPTH,)),
            ],
        ),
        out_shape=jax.ShapeDtypeStruct((B, NCLASS), jnp.float32),
    )(idx, emb, Wv, Wo, W1, W2, pos0, ln1_g, ln1_b, ln2_g, ln2_b,
      b1, b2, Wcls, bcls.reshape(1, NCLASS))
    return out


# speedup vs baseline: 413.4835x; 1.0181x over previous
import jax
import jax.numpy as jnp
from jax.experimental import pallas as pl
from jax.experimental.pallas import tpu as pltpu

DIM = 768
DEPTH = 2
NCLASS = 16
NSPLIT = 4


def _body(idx_ref, emb_any, Wv_any, Wo_any, W1_any, W2_any,
          pos_v, ln1g, ln1b, ln2g, ln2b, b1v, b2v, Wclsv, bclsv,
          out_ref, e_v, Wv_v, Wo_v, W1_v, W2_v, sems):
    B = e_v.shape[0]

    ecp = [pltpu.make_async_copy(emb_any.at[pl.ds(idx_ref[b], 1), :],
                                 e_v.at[pl.ds(b, 1), :], sems.at[b])
           for b in range(B)]
    sem_i = [B]

    def chunked(src, dst, l):
        rows = src.shape[1]
        cs = []
        for c in range(NSPLIT):
            r0, r1 = c * rows // NSPLIT, (c + 1) * rows // NSPLIT
            cs.append(pltpu.make_async_copy(
                src.at[l, pl.ds(r0, r1 - r0), :],
                dst.at[l, pl.ds(r0, r1 - r0), :], sems.at[sem_i[0]]))
            sem_i[0] += 1
        return cs

    wcp = []
    for l in range(DEPTH):
        wcp.append([chunked(Wv_any, Wv_v, l), chunked(Wo_any, Wo_v, l),
                    chunked(W1_any, W1_v, l), chunked(W2_any, W2_v, l)])
    for c in ecp:
        c.start()
    for cs in wcp:
        for ch in cs:
            for c in ch:
                c.start()

    def ln(v, g, bb):
        mu = jnp.mean(v, axis=-1, keepdims=True)
        var = jnp.mean((v - mu) ** 2, axis=-1, keepdims=True)
        return (v - mu) / jnp.sqrt(var + 1e-5) * g + bb

    for c in ecp:
        c.wait()
    h = e_v[...] + pos_v[...]

    for l in range(DEPTH):
        sl = pl.ds(l, 1)
        xln = ln(h, ln1g[sl, :], ln1b[sl, :])
        for c in wcp[l][0]:
            c.wait()
        a = jnp.dot(xln, Wv_v[l], preferred_element_type=jnp.float32)
        for c in wcp[l][1]:
            c.wait()
        a = jnp.dot(a, Wo_v[l], preferred_element_type=jnp.float32)
        h = h + a
        fln = ln(h, ln2g[sl, :], ln2b[sl, :])
        for c in wcp[l][2]:
            c.wait()
        f = jax.nn.gelu(jnp.dot(fln, W1_v[l], preferred_element_type=jnp.float32)
                        + b1v[sl, :])
        for c in wcp[l][3]:
            c.wait()
        f = jnp.dot(f, W2_v[l], preferred_element_type=jnp.float32) + b2v[sl, :]
        h = h + f

    out_ref[...] = jnp.dot(h, Wclsv[...],
                           preferred_element_type=jnp.float32) + bclsv[...]


def kernel(x, emb, pos, ln1_g, ln1_b, Wqk, Wv, Wo, ln2_g, ln2_b,
           W1, b1, W2, b2, rot, Wcls, bcls):
    B = x.shape[0]
    idx = x[:, 0]
    pos0 = pos[0:1, :]

    full = lambda shp: pl.BlockSpec(shp, lambda i, idx: tuple(0 for _ in shp))
    any_spec = pl.BlockSpec(memory_space=pl.MemorySpace.ANY)

    out = pl.pallas_call(
        _body,
        grid_spec=pltpu.PrefetchScalarGridSpec(
            num_scalar_prefetch=1,
            grid=(1,),
            in_specs=[
                any_spec,
                any_spec,
                any_spec,
                any_spec,
                any_spec,
                full((1, DIM)),
                full((DEPTH, DIM)),
                full((DEPTH, DIM)),
                full((DEPTH, DIM)),
                full((DEPTH, DIM)),
                full((DEPTH, 4 * DIM)),
                full((DEPTH, DIM)),
                full((DIM, NCLASS)),
                full((1, NCLASS)),
            ],
            out_specs=full((B, NCLASS)),
            scratch_shapes=[
                pltpu.VMEM((B, DIM), jnp.float32),
                pltpu.VMEM((DEPTH, DIM, DIM), jnp.float32),
                pltpu.VMEM((DEPTH, DIM, DIM), jnp.float32),
                pltpu.VMEM((DEPTH, DIM, 4 * DIM), jnp.float32),
                pltpu.VMEM((DEPTH, 4 * DIM, DIM), jnp.float32),
                pltpu.SemaphoreType.DMA((B + 4 * DEPTH * NSPLIT,)),
            ],
        ),
        out_shape=jax.ShapeDtypeStruct((B, NCLASS), jnp.float32),
    )(idx, emb, Wv, Wo, W1, W2, pos0, ln1_g, ln1_b, ln2_g, ln2_b,
      b1, b2, Wcls, bcls.reshape(1, NCLASS))
    return out


# speedup vs baseline: 447.4739x; 1.0822x over previous
import jax
import jax.numpy as jnp
from jax.experimental import pallas as pl
from jax.experimental.pallas import tpu as pltpu

DIM = 768
DEPTH = 2
NCLASS = 16
NSPLIT = 4


def _body(idx_ref, emb_any, Wv_any, Wo_any, W1_any, W2_any,
          pos_any, ln1g_any, ln1b_any, ln2g_any, ln2b_any,
          b1_any, b2_any, Wcls_any, bcls_any,
          out_ref, e_v, Wv_v, Wo_v, W1_v, W2_v,
          pos_v, ln1g_v, ln1b_v, ln2g_v, ln2b_v, b1_v, b2_v, Wcls_v, bcls_v,
          sems):
    B = e_v.shape[0]
    sem_i = [0]

    def copy(src, dst):
        c = pltpu.make_async_copy(src, dst, sems.at[sem_i[0]])
        sem_i[0] += 1
        c.start()
        return c

    ecp = [copy(emb_any.at[pl.ds(idx_ref[b], 1), :], e_v.at[pl.ds(b, 1), :])
           for b in range(B)]
    c_pos = copy(pos_any.at[pl.ds(0, 1), :], pos_v)
    c_ln1g = copy(ln1g_any, ln1g_v)
    c_ln1b = copy(ln1b_any, ln1b_v)
    c_ln2g = copy(ln2g_any, ln2g_v)
    c_ln2b = copy(ln2b_any, ln2b_v)
    c_b1 = copy(b1_any, b1_v)
    c_b2 = copy(b2_any, b2_v)
    c_wcls = copy(Wcls_any, Wcls_v)
    c_bcls = copy(bcls_any, bcls_v)

    def chunked(src, dst, l):
        rows = src.shape[1]
        cs = []
        for c in range(NSPLIT):
            r0, r1 = c * rows // NSPLIT, (c + 1) * rows // NSPLIT
            cs.append(copy(src.at[l, pl.ds(r0, r1 - r0), :],
                           dst.at[l, pl.ds(r0, r1 - r0), :]))
        return cs

    wcp = [[chunked(Wv_any, Wv_v, l), chunked(Wo_any, Wo_v, l),
            chunked(W1_any, W1_v, l), chunked(W2_any, W2_v, l)]
           for l in range(DEPTH)]

    def ln(v, g, bb):
        mu = jnp.mean(v, axis=-1, keepdims=True)
        var = jnp.mean((v - mu) ** 2, axis=-1, keepdims=True)
        return (v - mu) / jnp.sqrt(var + 1e-5) * g + bb

    for c in ecp:
        c.wait()
    c_pos.wait()
    h = e_v[...] + pos_v[...]

    c_ln1g.wait()
    c_ln1b.wait()
    c_ln2g.wait()
    c_ln2b.wait()
    c_b1.wait()
    c_b2.wait()
    for l in range(DEPTH):
        sl = pl.ds(l, 1)
        xln = ln(h, ln1g_v[sl, :], ln1b_v[sl, :])
        for c in wcp[l][0]:
            c.wait()
        a = jnp.dot(xln, Wv_v[l], preferred_element_type=jnp.float32)
        for c in wcp[l][1]:
            c.wait()
        a = jnp.dot(a, Wo_v[l], preferred_element_type=jnp.float32)
        h = h + a
        fln = ln(h, ln2g_v[sl, :], ln2b_v[sl, :])
        for c in wcp[l][2]:
            c.wait()
        f = jax.nn.gelu(jnp.dot(fln, W1_v[l], preferred_element_type=jnp.float32)
                        + b1_v[sl, :])
        for c in wcp[l][3]:
            c.wait()
        f = jnp.dot(f, W2_v[l], preferred_element_type=jnp.float32) + b2_v[sl, :]
        h = h + f

    c_wcls.wait()
    c_bcls.wait()
    out_ref[...] = jnp.dot(h, Wcls_v[...],
                           preferred_element_type=jnp.float32) + bcls_v[...]


def kernel(x, emb, pos, ln1_g, ln1_b, Wqk, Wv, Wo, ln2_g, ln2_b,
           W1, b1, W2, b2, rot, Wcls, bcls):
    B = x.shape[0]
    idx = x[:, 0]

    any_spec = pl.BlockSpec(memory_space=pl.MemorySpace.ANY)
    v2 = lambda r, c: pltpu.VMEM((r, c), jnp.float32)

    out = pl.pallas_call(
        _body,
        grid_spec=pltpu.PrefetchScalarGridSpec(
            num_scalar_prefetch=1,
            grid=(1,),
            in_specs=[any_spec] * 14,
            out_specs=pl.BlockSpec((B, NCLASS), lambda i, idx: (0, 0)),
            scratch_shapes=[
                v2(B, DIM),
                pltpu.VMEM((DEPTH, DIM, DIM), jnp.float32),
                pltpu.VMEM((DEPTH, DIM, DIM), jnp.float32),
                pltpu.VMEM((DEPTH, DIM, 4 * DIM), jnp.float32),
                pltpu.VMEM((DEPTH, 4 * DIM, DIM), jnp.float32),
                v2(1, DIM),
                v2(DEPTH, DIM), v2(DEPTH, DIM),
                v2(DEPTH, DIM), v2(DEPTH, DIM),
                v2(DEPTH, 4 * DIM), v2(DEPTH, DIM),
                v2(DIM, NCLASS), v2(1, NCLASS),
                pltpu.SemaphoreType.DMA((B + 10 + 4 * DEPTH * NSPLIT,)),
            ],
        ),
        out_shape=jax.ShapeDtypeStruct((B, NCLASS), jnp.float32),
    )(idx, emb, Wv, Wo, W1, W2, pos, ln1_g, ln1_b, ln2_g, ln2_b,
      b1, b2, Wcls, bcls.reshape(1, NCLASS))
    return out


# speedup vs baseline: 484.3792x; 1.0825x over previous
import jax
import jax.numpy as jnp
from jax.experimental import pallas as pl
from jax.experimental.pallas import tpu as pltpu

DIM = 768
DEPTH = 2
NCLASS = 16
NSPLIT = 4


def _body(idx_ref, emb_any, Wv_any, Wo_any, W1_any, W2_any,
          pos_any, ln1g_any, ln1b_any, ln2g_any, ln2b_any,
          b1_any, b2_any, Wcls_any, bcls_any,
          out_ref, e_v, Wv_v, Wo_v, W1_v, W2_v,
          pos_v, ln1g_v, ln1b_v, ln2g_v, ln2b_v, b1_v, b2_v, Wcls_v, bcls_v,
          sems):
    B = e_v.shape[0]
    sem_i = [0]

    def copy(src, dst):
        c = pltpu.make_async_copy(src, dst, sems.at[sem_i[0]])
        sem_i[0] += 1
        c.start()
        return c

    ecp = [copy(emb_any.at[pl.ds(idx_ref[b, 0], 1), :], e_v.at[pl.ds(b, 1), :])
           for b in range(B)]
    c_pos = copy(pos_any.at[pl.ds(0, 1), :], pos_v)
    c_ln1g = copy(ln1g_any, ln1g_v)
    c_ln1b = copy(ln1b_any, ln1b_v)
    c_ln2g = copy(ln2g_any, ln2g_v)
    c_ln2b = copy(ln2b_any, ln2b_v)
    c_b1 = copy(b1_any, b1_v)
    c_b2 = copy(b2_any, b2_v)
    c_wcls = copy(Wcls_any, Wcls_v)
    c_bcls = copy(bcls_any, bcls_v)

    def chunked(src, dst, l):
        rows = src.shape[1]
        cs = []
        for c in range(NSPLIT):
            r0, r1 = c * rows // NSPLIT, (c + 1) * rows // NSPLIT
            cs.append(copy(src.at[l, pl.ds(r0, r1 - r0), :],
                           dst.at[l, pl.ds(r0, r1 - r0), :]))
        return cs

    wcp = [[chunked(Wv_any, Wv_v, l), chunked(Wo_any, Wo_v, l),
            chunked(W1_any, W1_v, l), chunked(W2_any, W2_v, l)]
           for l in range(DEPTH)]

    def ln(v, g, bb):
        mu = jnp.mean(v, axis=-1, keepdims=True)
        var = jnp.mean((v - mu) ** 2, axis=-1, keepdims=True)
        return (v - mu) / jnp.sqrt(var + 1e-5) * g + bb

    for c in ecp:
        c.wait()
    c_pos.wait()
    h = e_v[...] + pos_v[...]

    c_ln1g.wait()
    c_ln1b.wait()
    c_ln2g.wait()
    c_ln2b.wait()
    c_b1.wait()
    c_b2.wait()
    for l in range(DEPTH):
        sl = pl.ds(l, 1)
        xln = ln(h, ln1g_v[sl, :], ln1b_v[sl, :])
        for c in wcp[l][0]:
            c.wait()
        a = jnp.dot(xln, Wv_v[l], preferred_element_type=jnp.float32)
        for c in wcp[l][1]:
            c.wait()
        a = jnp.dot(a, Wo_v[l], preferred_element_type=jnp.float32)
        h = h + a
        fln = ln(h, ln2g_v[sl, :], ln2b_v[sl, :])
        for c in wcp[l][2]:
            c.wait()
        f = jax.nn.gelu(jnp.dot(fln, W1_v[l], preferred_element_type=jnp.float32)
                        + b1_v[sl, :])
        acc = b2_v[sl, :]
        rows = W2_v.shape[1]
        for c in range(NSPLIT):
            r0, r1 = c * rows // NSPLIT, (c + 1) * rows // NSPLIT
            wcp[l][3][c].wait()
            acc = acc + jnp.dot(f[:, r0:r1], W2_v[l, r0:r1, :],
                                preferred_element_type=jnp.float32)
        h = h + acc

    c_wcls.wait()
    c_bcls.wait()
    out_ref[...] = jnp.dot(h, Wcls_v[...],
                           preferred_element_type=jnp.float32) + bcls_v[...]


def kernel(x, emb, pos, ln1_g, ln1_b, Wqk, Wv, Wo, ln2_g, ln2_b,
           W1, b1, W2, b2, rot, Wcls, bcls):
    B = x.shape[0]

    any_spec = pl.BlockSpec(memory_space=pl.MemorySpace.ANY)
    v2 = lambda r, c: pltpu.VMEM((r, c), jnp.float32)

    out = pl.pallas_call(
        _body,
        grid_spec=pltpu.PrefetchScalarGridSpec(
            num_scalar_prefetch=1,
            grid=(1,),
            in_specs=[any_spec] * 14,
            out_specs=pl.BlockSpec((B, NCLASS), lambda i, idx: (0, 0)),
            scratch_shapes=[
                v2(B, DIM),
                pltpu.VMEM((DEPTH, DIM, DIM), jnp.float32),
                pltpu.VMEM((DEPTH, DIM, DIM), jnp.float32),
                pltpu.VMEM((DEPTH, DIM, 4 * DIM), jnp.float32),
                pltpu.VMEM((DEPTH, 4 * DIM, DIM), jnp.float32),
                v2(1, DIM),
                v2(DEPTH, DIM), v2(DEPTH, DIM),
                v2(DEPTH, DIM), v2(DEPTH, DIM),
                v2(DEPTH, 4 * DIM), v2(DEPTH, DIM),
                v2(DIM, NCLASS), v2(1, NCLASS),
                pltpu.SemaphoreType.DMA((B + 10 + 4 * DEPTH * NSPLIT,)),
            ],
        ),
        out_shape=jax.ShapeDtypeStruct((B, NCLASS), jnp.float32),
    )(x, emb, Wv, Wo, W1, W2, pos, ln1_g, ln1_b, ln2_g, ln2_b,
      b1, b2, Wcls, bcls.reshape(1, NCLASS))
    return out
